# Initial kernel scaffold; baseline (speedup 1.0000x reference)
import jax
import jax.numpy as jnp
from jax.experimental import pallas as pl


def kernel(X, W_enc1, W_enc2, W_str1, W_att1, W_att2, edge_index):
    raise NotImplementedError("write your pallas kernel here")


# speedup vs baseline: 15.8975x; 15.8975x over previous
import functools

import jax
import jax.numpy as jnp
from jax import lax
from jax.experimental import pallas as pl
from jax.experimental.pallas import tpu as pltpu
from jax.experimental.pallas import tpu_sc as plsc

N = 10000
E = 320000
NC = 2
NS = 16
NW = NC * NS
EPT = E // NW
CHUNK = 80
NCHUNK = EPT // CHUNK
RPT = N // NS
DRAIN = 632

_f32 = jnp.float32


def _mesh():
    return plsc.VectorSubcoreMesh(core_axis_name="c", subcore_axis_name="s")


@functools.partial(
    pl.kernel,
    out_type=jax.ShapeDtypeStruct((NW, N), _f32),
    mesh=_mesh(),
    scratch_types=[
        pltpu.VMEM((EPT,), jnp.int32),
        pltpu.VMEM((N,), _f32),
    ],
    compiler_params=pltpu.CompilerParams(
        needs_layout_passes=False, use_tc_tiling_on_sc=False),
)
def _deg_kernel(dst_hbm, zeros_hbm, out_hbm, dst_v, hist_v):
    cid = lax.axis_index("c")
    sid = lax.axis_index("s")
    wid = cid * NS + sid
    pltpu.sync_copy(dst_hbm.at[wid], dst_v)
    pltpu.sync_copy(zeros_hbm, hist_v)
    ones = jnp.full((16,), 1.0, _f32)

    def body(k, carry):
        idx = dst_v[pl.ds(k * 16, 16)]
        plsc.addupdate_scatter(hist_v, [idx], ones)
        return carry

    lax.fori_loop(0, EPT // 16, body, 0)
    pltpu.sync_copy(hist_v, out_hbm.at[wid])


def _make_prop(F):
    @functools.partial(
        pl.kernel,
        out_type=jax.ShapeDtypeStruct((NC, N, F), _f32),
        mesh=_mesh(),
        scratch_types=[
            pltpu.VMEM((NCHUNK, CHUNK), jnp.int32),
            pltpu.VMEM((NCHUNK, CHUNK), jnp.int32),
            pltpu.VMEM((CHUNK, F), _f32),
            pltpu.VMEM_SHARED((N, F), _f32),
        ],
        compiler_params=pltpu.CompilerParams(use_tc_tiling_on_sc=False),
    )
    def prop(g_hbm, src_hbm, dst_hbm, zeros_hbm, out_hbm, si_v, di_v, rows_v, acc_s):
        cid = lax.axis_index("c")
        sid = lax.axis_index("s")
        wid = cid * NS + sid
        pltpu.sync_copy(src_hbm.at[wid], si_v)
        pltpu.sync_copy(dst_hbm.at[wid], di_v)
        start = pl.multiple_of((sid * RPT) // 8 * 8, 8)
        pltpu.sync_copy(zeros_hbm, acc_s.at[pl.ds(start, DRAIN)])
        plsc.subcore_barrier()

        def body(k, carry):
            pltpu.sync_copy(g_hbm.at[si_v.at[k]], rows_v)
            pltpu.sync_copy(rows_v, acc_s.at[di_v.at[k]], add=True)
            return carry

        lax.fori_loop(0, NCHUNK, body, 0)
        plsc.subcore_barrier()
        pltpu.sync_copy(acc_s.at[pl.ds(start, DRAIN)],
                        out_hbm.at[cid, pl.ds(start, DRAIN)])

    return prop


_PROP = {F: _make_prop(F) for F in (32, 64, 128)}


def _dinv_from_partials(degs):
    def body(d_ref, o_ref):
        o_ref[...] = lax.rsqrt(1.0 + jnp.sum(d_ref[...], axis=0, keepdims=True))

    row = pl.pallas_call(
        body,
        out_shape=jax.ShapeDtypeStruct((1, N), _f32),
    )(degs)
    return row.reshape(N, 1)


def _mm_scale(x, w, dinv):
    bm = 2000
    fin, f = w.shape

    def body(x_ref, w_ref, d_ref, o_ref):
        o_ref[...] = d_ref[...] * jnp.dot(
            x_ref[...], w_ref[...], preferred_element_type=_f32)

    return pl.pallas_call(
        body,
        grid=(N // bm,),
        in_specs=[
            pl.BlockSpec((bm, fin), lambda i: (i, 0)),
            pl.BlockSpec((fin, f), lambda i: (0, 0)),
            pl.BlockSpec((bm, 1), lambda i: (i, 0)),
        ],
        out_specs=pl.BlockSpec((bm, f), lambda i: (i, 0)),
        out_shape=jax.ShapeDtypeStruct((N, f), _f32),
    )(x, w, dinv)


def _combine(s, g, dinv):
    bm = 2000
    f = g.shape[1]

    def body(s_ref, g_ref, d_ref, o_ref):
        t = s_ref[0] + s_ref[1] + g_ref[...]
        o_ref[...] = jnp.maximum(d_ref[...] * t, 0.0)

    return pl.pallas_call(
        body,
        grid=(N // bm,),
        in_specs=[
            pl.BlockSpec((NC, bm, f), lambda i: (0, i, 0)),
            pl.BlockSpec((bm, f), lambda i: (i, 0)),
            pl.BlockSpec((bm, 1), lambda i: (i, 0)),
        ],
        out_specs=pl.BlockSpec((bm, f), lambda i: (i, 0)),
        out_shape=jax.ShapeDtypeStruct((N, f), _f32),
    )(s, g, dinv)


def _gram_sigmoid(a):
    bm = 400
    f = a.shape[1]

    def body(a_ref, b_ref, o_ref):
        y = lax.dot_general(a_ref[...], b_ref[...],
                            (((1,), (1,)), ((), ())),
                            preferred_element_type=_f32)
        o_ref[...] = jax.nn.sigmoid(y)

    return pl.pallas_call(
        body,
        grid=(N // bm,),
        in_specs=[
            pl.BlockSpec((bm, f), lambda i: (i, 0)),
            pl.BlockSpec((N, f), lambda i: (0, 0)),
        ],
        out_specs=pl.BlockSpec((bm, N), lambda i: (i, 0)),
        out_shape=jax.ShapeDtypeStruct((N, N), _f32),
        compiler_params=pltpu.CompilerParams(
            dimension_semantics=("arbitrary",)),
    )(a, a)


def kernel(X, W_enc1, W_enc2, W_str1, W_att1, W_att2, edge_index):
    src3 = edge_index[0].reshape(NW, NCHUNK, CHUNK)
    dst3 = edge_index[1].reshape(NW, NCHUNK, CHUNK)
    dst2 = edge_index[1].reshape(NW, EPT)
    zeros_n = jnp.zeros((N,), _f32)

    degs = _deg_kernel(dst2, zeros_n)
    dinv = _dinv_from_partials(degs)

    def layer(x, w):
        f = w.shape[1]
        g = _mm_scale(x, w, dinv)
        s = _PROP[f](g, src3, dst3, jnp.zeros((DRAIN, f), _f32))
        return _combine(s, g, dinv)

    h = layer(X, W_enc1)
    h = layer(h, W_enc2)
    a0 = layer(h, W_str1)
    att = layer(h, W_att1)
    att = layer(att, W_att2)
    A = _gram_sigmoid(a0)
    return (att, A)


# speedup vs baseline: 18.1257x; 1.1402x over previous
import functools

import jax
import jax.numpy as jnp
from jax import lax
from jax.experimental import pallas as pl
from jax.experimental.pallas import tpu as pltpu
from jax.experimental.pallas import tpu_sc as plsc

N = 10000
E = 320000
NC = 2
NS = 16
NW = NC * NS
EPT = E // NW
NBUF = 5
RPT = N // NS
DRAIN = 632

_f32 = jnp.float32


def _mesh():
    return plsc.VectorSubcoreMesh(core_axis_name="c", subcore_axis_name="s")


@functools.partial(
    pl.kernel,
    out_type=jax.ShapeDtypeStruct((NW, N), _f32),
    mesh=_mesh(),
    scratch_types=[
        pltpu.VMEM((EPT,), jnp.int32),
        pltpu.VMEM((N,), _f32),
    ],
    compiler_params=pltpu.CompilerParams(
        needs_layout_passes=False, use_tc_tiling_on_sc=False),
)
def _deg_kernel(dst_hbm, zeros_hbm, out_hbm, dst_v, hist_v):
    cid = lax.axis_index("c")
    sid = lax.axis_index("s")
    wid = cid * NS + sid
    pltpu.sync_copy(dst_hbm.at[wid], dst_v)
    pltpu.sync_copy(zeros_hbm, hist_v)
    ones = jnp.full((16,), 1.0, _f32)

    def body(k, carry):
        idx = dst_v[pl.ds(k * 16, 16)]
        plsc.addupdate_scatter(hist_v, [idx], ones)
        return carry

    lax.fori_loop(0, EPT // 16, body, 0)
    pltpu.sync_copy(hist_v, out_hbm.at[wid])


def _make_prop(F):
    CHUNK = 40
    NCHUNK = EPT // CHUNK

    @functools.partial(
        pl.kernel,
        out_type=jax.ShapeDtypeStruct((NC, N, F), _f32),
        mesh=_mesh(),
        scratch_types=[
            pltpu.VMEM((NCHUNK, CHUNK), jnp.int32),
            pltpu.VMEM((NCHUNK, CHUNK), jnp.int32),
            pltpu.VMEM((2, CHUNK, F), _f32),
            pltpu.VMEM_SHARED((N, F), _f32),
            pltpu.SemaphoreType.DMA,
            pltpu.SemaphoreType.DMA,
        ],
        compiler_params=pltpu.CompilerParams(use_tc_tiling_on_sc=False),
    )
    def prop(g_hbm, src_hbm, dst_hbm, zeros_hbm, out_hbm,
             si_v, di_v, rows_v, acc_s, sem0, sem1):
        cid = lax.axis_index("c")
        sid = lax.axis_index("s")
        wid = cid * NS + sid
        pltpu.sync_copy(src_hbm.at[wid], si_v)
        pltpu.sync_copy(dst_hbm.at[wid], di_v)
        start = pl.multiple_of((sid * RPT) // 8 * 8, 8)
        pltpu.sync_copy(zeros_hbm, acc_s.at[pl.ds(start, DRAIN)])
        plsc.subcore_barrier()

        sems = (sem0, sem1)

        def gather(k, b):
            pltpu.async_copy(g_hbm.at[si_v.at[k]], rows_v.at[b], sems[b])

        def gather_wait(k, b):
            pltpu.make_async_copy(g_hbm.at[si_v.at[k]], rows_v.at[b],
                                  sems[b]).wait()

        gather(0, 0)

        def group(o, carry):
            for j in range(2):
                k = o * 2 + j

                @pl.when(k + 1 < NCHUNK)
                def _():
                    gather(k + 1, 1 - j)

                gather_wait(k, j)
                pltpu.sync_copy(rows_v.at[j], acc_s.at[di_v.at[k]], add=True)
            return carry

        lax.fori_loop(0, NCHUNK // 2, group, 0)
        plsc.subcore_barrier()
        pltpu.sync_copy(acc_s.at[pl.ds(start, DRAIN)],
                        out_hbm.at[cid, pl.ds(start, DRAIN)])

    return prop


_PROP = {F: _make_prop(F) for F in (32, 64, 128)}


def _dinv_from_partials(degs):
    def body(d_ref, o_ref):
        o_ref[...] = lax.rsqrt(1.0 + jnp.sum(d_ref[...], axis=0, keepdims=True))

    row = pl.pallas_call(
        body,
        out_shape=jax.ShapeDtypeStruct((1, N), _f32),
    )(degs)
    return row.reshape(N, 1)


def _mm_scale(x, w, dinv):
    bm = 2000
    fin, f = w.shape

    def body(x_ref, w_ref, d_ref, o_ref):
        o_ref[...] = d_ref[...] * jnp.dot(
            x_ref[...], w_ref[...], preferred_element_type=_f32)

    return pl.pallas_call(
        body,
        grid=(N // bm,),
        in_specs=[
            pl.BlockSpec((bm, fin), lambda i: (i, 0)),
            pl.BlockSpec((fin, f), lambda i: (0, 0)),
            pl.BlockSpec((bm, 1), lambda i: (i, 0)),
        ],
        out_specs=pl.BlockSpec((bm, f), lambda i: (i, 0)),
        out_shape=jax.ShapeDtypeStruct((N, f), _f32),
    )(x, w, dinv)


def _combine(s, g, dinv):
    bm = 2000
    f = g.shape[1]

    def body(s_ref, g_ref, d_ref, o_ref):
        t = s_ref[0] + s_ref[1] + g_ref[...]
        o_ref[...] = jnp.maximum(d_ref[...] * t, 0.0)

    return pl.pallas_call(
        body,
        grid=(N // bm,),
        in_specs=[
            pl.BlockSpec((NC, bm, f), lambda i: (0, i, 0)),
            pl.BlockSpec((bm, f), lambda i: (i, 0)),
            pl.BlockSpec((bm, 1), lambda i: (i, 0)),
        ],
        out_specs=pl.BlockSpec((bm, f), lambda i: (i, 0)),
        out_shape=jax.ShapeDtypeStruct((N, f), _f32),
    )(s, g, dinv)


def _gram_sigmoid(a):
    bm = 400
    f = a.shape[1]

    def body(a_ref, b_ref, o_ref):
        y = lax.dot_general(a_ref[...], b_ref[...],
                            (((1,), (1,)), ((), ())),
                            preferred_element_type=_f32)
        o_ref[...] = jax.nn.sigmoid(y)

    return pl.pallas_call(
        body,
        grid=(N // bm,),
        in_specs=[
            pl.BlockSpec((bm, f), lambda i: (i, 0)),
            pl.BlockSpec((N, f), lambda i: (0, 0)),
        ],
        out_specs=pl.BlockSpec((bm, N), lambda i: (i, 0)),
        out_shape=jax.ShapeDtypeStruct((N, N), _f32),
        compiler_params=pltpu.CompilerParams(
            dimension_semantics=("arbitrary",)),
    )(a, a)


def kernel(X, W_enc1, W_enc2, W_str1, W_att1, W_att2, edge_index):
    dst2 = edge_index[1].reshape(NW, EPT)
    zeros_n = jnp.zeros((N,), _f32)

    degs = _deg_kernel(dst2, zeros_n)
    dinv = _dinv_from_partials(degs)

    def layer(x, w):
        f = w.shape[1]
        ch = 40
        src3 = edge_index[0].reshape(NW, EPT // ch, ch)
        dst3 = edge_index[1].reshape(NW, EPT // ch, ch)
        g = _mm_scale(x, w, dinv)
        s = _PROP[f](g, src3, dst3, jnp.zeros((DRAIN, f), _f32))
        return _combine(s, g, dinv)

    h = layer(X, W_enc1)
    h = layer(h, W_enc2)
    a0 = layer(h, W_str1)
    att = layer(h, W_att1)
    att = layer(att, W_att2)
    A = _gram_sigmoid(a0)
    return (att, A)


# speedup vs baseline: 24.5456x; 1.3542x over previous
import functools

import jax
import jax.numpy as jnp
from jax import lax
from jax.experimental import pallas as pl
from jax.experimental.pallas import tpu as pltpu
from jax.experimental.pallas import tpu_sc as plsc

N = 10000
E = 320000
NFEAT = 128
NHID1 = 64
NC = 2
NS = 16
NW = NC * NS
EPT = E // NW
NBUF = 5
RPT = N // NS
DRAIN = 632

_f32 = jnp.float32


def _mesh():
    return plsc.VectorSubcoreMesh(core_axis_name="c", subcore_axis_name="s")


@functools.partial(
    pl.kernel,
    out_type=jax.ShapeDtypeStruct((NW, N), _f32),
    mesh=_mesh(),
    scratch_types=[
        pltpu.VMEM((EPT,), jnp.int32),
        pltpu.VMEM((N,), _f32),
    ],
    compiler_params=pltpu.CompilerParams(
        needs_layout_passes=False, use_tc_tiling_on_sc=False),
)
def _deg_kernel(dst_hbm, zeros_hbm, out_hbm, dst_v, hist_v):
    cid = lax.axis_index("c")
    sid = lax.axis_index("s")
    wid = cid * NS + sid
    pltpu.sync_copy(dst_hbm.at[wid], dst_v)
    pltpu.sync_copy(zeros_hbm, hist_v)
    ones = jnp.full((16,), 1.0, _f32)

    def body(k, carry):
        idx = dst_v[pl.ds(k * 16, 16)]
        plsc.addupdate_scatter(hist_v, [idx], ones)
        return carry

    lax.fori_loop(0, EPT // 16, body, 0)
    pltpu.sync_copy(hist_v, out_hbm.at[wid])


def _make_prop(F):
    CHUNK = 80
    NCHUNK = EPT // CHUNK

    @functools.partial(
        pl.kernel,
        out_type=jax.ShapeDtypeStruct((NC, N, F), _f32),
        mesh=_mesh(),
        scratch_types=[
            pltpu.VMEM((NCHUNK, CHUNK), jnp.int32),
            pltpu.VMEM((NCHUNK, CHUNK), jnp.int32),
            pltpu.VMEM((2, CHUNK, F), _f32),
            pltpu.VMEM_SHARED((N, F), _f32),
            pltpu.SemaphoreType.DMA,
            pltpu.SemaphoreType.DMA,
        ],
        compiler_params=pltpu.CompilerParams(use_tc_tiling_on_sc=False),
    )
    def prop(g_hbm, src_hbm, dst_hbm, zeros_hbm, out_hbm,
             si_v, di_v, rows_v, acc_s, sem0, sem1):
        cid = lax.axis_index("c")
        sid = lax.axis_index("s")
        wid = cid * NS + sid
        pltpu.sync_copy(src_hbm.at[wid], si_v)
        pltpu.sync_copy(dst_hbm.at[wid], di_v)
        start = pl.multiple_of((sid * RPT) // 8 * 8, 8)
        pltpu.sync_copy(zeros_hbm, acc_s.at[pl.ds(start, DRAIN)])
        plsc.subcore_barrier()

        sems = (sem0, sem1)

        def gather(k, b):
            pltpu.async_copy(g_hbm.at[si_v.at[k]], rows_v.at[b], sems[b])

        def gather_wait(k, b):
            pltpu.make_async_copy(g_hbm.at[si_v.at[k]], rows_v.at[b],
                                  sems[b]).wait()

        gather(0, 0)

        def group(o, carry):
            for j in range(2):
                k = o * 2 + j
                gather(k + 1, 1 - j)
                gather_wait(k, j)
                pltpu.sync_copy(rows_v.at[j], acc_s.at[di_v.at[k]], add=True)
            return carry

        lax.fori_loop(0, NCHUNK // 2, group, 0)
        gather_wait(NCHUNK - 1, 0)
        pltpu.sync_copy(rows_v.at[0], acc_s.at[di_v.at[NCHUNK - 1]], add=True)
        plsc.subcore_barrier()
        pltpu.sync_copy(acc_s.at[pl.ds(start, DRAIN)],
                        out_hbm.at[cid, pl.ds(start, DRAIN)])

    return prop


_PROP = {F: _make_prop(F) for F in (32, 64, 128)}


def _dinv_from_partials(degs):
    def body(d_ref, o_ref):
        o_ref[...] = lax.rsqrt(1.0 + jnp.sum(d_ref[...], axis=0, keepdims=True))

    row = pl.pallas_call(
        body,
        out_shape=jax.ShapeDtypeStruct((1, N), _f32),
    )(degs)
    return row.reshape(N, 1)


def _mm_scale(x, w, dinv):
    bm = 2000
    fin, f = w.shape

    def body(x_ref, w_ref, d_ref, o_ref):
        o_ref[...] = d_ref[...] * jnp.dot(
            x_ref[...], w_ref[...], preferred_element_type=_f32)

    return pl.pallas_call(
        body,
        grid=(N // bm,),
        in_specs=[
            pl.BlockSpec((bm, fin), lambda i: (i, 0)),
            pl.BlockSpec((fin, f), lambda i: (0, 0)),
            pl.BlockSpec((bm, 1), lambda i: (i, 0)),
        ],
        out_specs=pl.BlockSpec((bm, f), lambda i: (i, 0)),
        out_shape=jax.ShapeDtypeStruct((N, f), _f32),
    )(x, w, dinv)


def _combine(s, g, dinv):
    bm = 2000
    f = g.shape[1]

    def body(s_ref, g_ref, d_ref, o_ref):
        t = s_ref[0] + s_ref[1] + g_ref[...]
        o_ref[...] = jnp.maximum(d_ref[...] * t, 0.0)

    return pl.pallas_call(
        body,
        grid=(N // bm,),
        in_specs=[
            pl.BlockSpec((NC, bm, f), lambda i: (0, i, 0)),
            pl.BlockSpec((bm, f), lambda i: (i, 0)),
            pl.BlockSpec((bm, 1), lambda i: (i, 0)),
        ],
        out_specs=pl.BlockSpec((bm, f), lambda i: (i, 0)),
        out_shape=jax.ShapeDtypeStruct((N, f), _f32),
    )(s, g, dinv)


def _gram_sigmoid(a, fsub):
    bm = 400
    f = a.shape[1]

    def body(a_ref, b_ref, o_ref):
        y = lax.dot_general(a_ref[:, :fsub], b_ref[:, :fsub],
                            (((1,), (1,)), ((), ())),
                            preferred_element_type=_f32)
        o_ref[...] = jax.nn.sigmoid(y)

    return pl.pallas_call(
        body,
        grid=(N // bm,),
        in_specs=[
            pl.BlockSpec((bm, f), lambda i: (i, 0)),
            pl.BlockSpec((N, f), lambda i: (0, 0)),
        ],
        out_specs=pl.BlockSpec((bm, N), lambda i: (i, 0)),
        out_shape=jax.ShapeDtypeStruct((N, N), _f32),
        compiler_params=pltpu.CompilerParams(
            dimension_semantics=("arbitrary",)),
    )(a, a)


def kernel(X, W_enc1, W_enc2, W_str1, W_att1, W_att2, edge_index):
    dst2 = edge_index[1].reshape(NW, EPT)
    zeros_n = jnp.zeros((N,), _f32)

    degs = _deg_kernel(dst2, zeros_n)
    dinv = _dinv_from_partials(degs)

    def layer(x, w):
        f = w.shape[1]
        ch = 80
        src3 = edge_index[0].reshape(NW, EPT // ch, ch)
        dst3 = edge_index[1].reshape(NW, EPT // ch, ch)
        g = _mm_scale(x, w, dinv)
        s = _PROP[f](g, src3, dst3, jnp.zeros((DRAIN, f), _f32))
        return _combine(s, g, dinv)

    h = layer(X, W_enc1)
    h = layer(h, W_enc2)
    w34 = jnp.concatenate([W_str1, W_att1], axis=1)
    h34 = layer(h, w34)
    w5 = jnp.concatenate([jnp.zeros((NHID1, NFEAT), _f32), W_att2], axis=0)
    att = layer(h34, w5)
    A = _gram_sigmoid(h34, NHID1)
    return (att, A)


# speedup vs baseline: 24.5899x; 1.0018x over previous
import functools

import jax
import jax.numpy as jnp
from jax import lax
from jax.experimental import pallas as pl
from jax.experimental.pallas import tpu as pltpu
from jax.experimental.pallas import tpu_sc as plsc

N = 10000
E = 320000
NFEAT = 128
NHID1 = 64
NC = 2
NS = 16
NW = NC * NS
EPT = E // NW
NBUF = 5
RPT = N // NS
DRAIN = 632

_f32 = jnp.float32


def _mesh():
    return plsc.VectorSubcoreMesh(core_axis_name="c", subcore_axis_name="s")


@functools.partial(
    pl.kernel,
    out_type=jax.ShapeDtypeStruct((NW, N), _f32),
    mesh=_mesh(),
    scratch_types=[
        pltpu.VMEM((EPT,), jnp.int32),
        pltpu.VMEM((N,), _f32),
    ],
    compiler_params=pltpu.CompilerParams(
        needs_layout_passes=False, use_tc_tiling_on_sc=False),
)
def _deg_kernel(dst_hbm, zeros_hbm, out_hbm, dst_v, hist_v):
    cid = lax.axis_index("c")
    sid = lax.axis_index("s")
    wid = cid * NS + sid
    pltpu.sync_copy(dst_hbm.at[wid], dst_v)
    pltpu.sync_copy(zeros_hbm, hist_v)
    ones = jnp.full((16,), 1.0, _f32)

    def body(k, carry):
        idx = dst_v[pl.ds(k * 16, 16)]
        plsc.addupdate_scatter(hist_v, [idx], ones)
        return carry

    lax.fori_loop(0, EPT // 16, body, 0)
    pltpu.sync_copy(hist_v, out_hbm.at[wid])


def _make_prop(F):
    CHUNK = 80
    NCHUNK = EPT // CHUNK

    @functools.partial(
        pl.kernel,
        out_type=jax.ShapeDtypeStruct((NC, N, F), _f32),
        mesh=_mesh(),
        scratch_types=[
            pltpu.VMEM((NCHUNK, CHUNK), jnp.int32),
            pltpu.VMEM((NCHUNK, CHUNK), jnp.int32),
            pltpu.VMEM((2, CHUNK, F), _f32),
            pltpu.VMEM_SHARED((N, F), _f32),
            pltpu.SemaphoreType.DMA,
            pltpu.SemaphoreType.DMA,
            pltpu.SemaphoreType.DMA,
            pltpu.SemaphoreType.DMA,
        ],
        compiler_params=pltpu.CompilerParams(use_tc_tiling_on_sc=False),
    )
    def prop(g_hbm, src_hbm, dst_hbm, zeros_hbm, out_hbm,
             si_v, di_v, rows_v, acc_s, sem0, sem1, sem2, sem3):
        cid = lax.axis_index("c")
        sid = lax.axis_index("s")
        wid = cid * NS + sid
        pltpu.sync_copy(src_hbm.at[wid], si_v)
        pltpu.sync_copy(dst_hbm.at[wid], di_v)
        start = pl.multiple_of((sid * RPT) // 8 * 8, 8)
        pltpu.sync_copy(zeros_hbm, acc_s.at[pl.ds(start, DRAIN)])
        plsc.subcore_barrier()

        gsems = (sem0, sem1)
        ssems = (sem2, sem3)

        def gather(k, b):
            pltpu.async_copy(g_hbm.at[si_v.at[k]], rows_v.at[b], gsems[b])

        def gather_wait(k, b):
            pltpu.make_async_copy(g_hbm.at[si_v.at[k]], rows_v.at[b],
                                  gsems[b]).wait()

        def scatter(k, b):
            pltpu.async_copy(rows_v.at[b], acc_s.at[di_v.at[k]], ssems[b],
                             add=True)

        def scatter_wait(k, b):
            pltpu.make_async_copy(rows_v.at[b], acc_s.at[di_v.at[k]],
                                  ssems[b]).wait()

        gather(0, 0)

        def group(o, carry):
            for j in range(2):
                k = o * 2 + j
                if j == 0:
                    @pl.when(o >= 1)
                    def _():
                        scatter_wait(k - 1, 1)
                else:
                    scatter_wait(k - 1, 0)
                gather(k + 1, 1 - j)
                gather_wait(k, j)
                scatter(k, j)
            return carry

        lax.fori_loop(0, NCHUNK // 2, group, 0)
        gather_wait(NCHUNK - 1, 0)
        scatter_wait(NCHUNK - 2, 1)
        pltpu.sync_copy(rows_v.at[0], acc_s.at[di_v.at[NCHUNK - 1]], add=True)
        plsc.subcore_barrier()
        pltpu.sync_copy(acc_s.at[pl.ds(start, DRAIN)],
                        out_hbm.at[cid, pl.ds(start, DRAIN)])

    return prop


_PROP = {F: _make_prop(F) for F in (32, 64, 128)}


def _dinv_from_partials(degs):
    def body(d_ref, o_ref):
        o_ref[...] = lax.rsqrt(1.0 + jnp.sum(d_ref[...], axis=0, keepdims=True))

    row = pl.pallas_call(
        body,
        out_shape=jax.ShapeDtypeStruct((1, N), _f32),
    )(degs)
    return row.reshape(N, 1)


def _mm_scale(x, w, dinv):
    bm = 2000
    fin, f = w.shape

    def body(x_ref, w_ref, d_ref, o_ref):
        o_ref[...] = d_ref[...] * jnp.dot(
            x_ref[...], w_ref[...], preferred_element_type=_f32)

    return pl.pallas_call(
        body,
        grid=(N // bm,),
        in_specs=[
            pl.BlockSpec((bm, fin), lambda i: (i, 0)),
            pl.BlockSpec((fin, f), lambda i: (0, 0)),
            pl.BlockSpec((bm, 1), lambda i: (i, 0)),
        ],
        out_specs=pl.BlockSpec((bm, f), lambda i: (i, 0)),
        out_shape=jax.ShapeDtypeStruct((N, f), _f32),
    )(x, w, dinv)


def _combine(s, g, dinv):
    bm = 2000
    f = g.shape[1]

    def body(s_ref, g_ref, d_ref, o_ref):
        t = s_ref[0] + s_ref[1] + g_ref[...]
        o_ref[...] = jnp.maximum(d_ref[...] * t, 0.0)

    return pl.pallas_call(
        body,
        grid=(N // bm,),
        in_specs=[
            pl.BlockSpec((NC, bm, f), lambda i: (0, i, 0)),
            pl.BlockSpec((bm, f), lambda i: (i, 0)),
            pl.BlockSpec((bm, 1), lambda i: (i, 0)),
        ],
        out_specs=pl.BlockSpec((bm, f), lambda i: (i, 0)),
        out_shape=jax.ShapeDtypeStruct((N, f), _f32),
    )(s, g, dinv)


def _gram_sigmoid(a, fsub):
    bm = 400
    f = a.shape[1]

    def body(a_ref, b_ref, o_ref):
        y = lax.dot_general(a_ref[:, :fsub], b_ref[:, :fsub],
                            (((1,), (1,)), ((), ())),
                            preferred_element_type=_f32)
        o_ref[...] = jax.nn.sigmoid(y)

    return pl.pallas_call(
        body,
        grid=(N // bm,),
        in_specs=[
            pl.BlockSpec((bm, f), lambda i: (i, 0)),
            pl.BlockSpec((N, f), lambda i: (0, 0)),
        ],
        out_specs=pl.BlockSpec((bm, N), lambda i: (i, 0)),
        out_shape=jax.ShapeDtypeStruct((N, N), _f32),
        compiler_params=pltpu.CompilerParams(
            dimension_semantics=("arbitrary",)),
    )(a, a)


def kernel(X, W_enc1, W_enc2, W_str1, W_att1, W_att2, edge_index):
    dst2 = edge_index[1].reshape(NW, EPT)
    zeros_n = jnp.zeros((N,), _f32)

    degs = _deg_kernel(dst2, zeros_n)
    dinv = _dinv_from_partials(degs)

    def layer(x, w):
        f = w.shape[1]
        ch = 80
        src3 = edge_index[0].reshape(NW, EPT // ch, ch)
        dst3 = edge_index[1].reshape(NW, EPT // ch, ch)
        g = _mm_scale(x, w, dinv)
        s = _PROP[f](g, src3, dst3, jnp.zeros((DRAIN, f), _f32))
        return _combine(s, g, dinv)

    h = layer(X, W_enc1)
    h = layer(h, W_enc2)
    w34 = jnp.concatenate([W_str1, W_att1], axis=1)
    h34 = layer(h, w34)
    w5 = jnp.concatenate([jnp.zeros((NHID1, NFEAT), _f32), W_att2], axis=0)
    att = layer(h34, w5)
    A = _gram_sigmoid(h34, NHID1)
    return (att, A)
